# baseline (speedup 1.0000x reference)
import functools

import jax
import jax.numpy as jnp
from jax import lax
from jax.experimental import pallas as pl
from jax.experimental.pallas import tpu as pltpu
from jax.experimental.pallas import tpu_sc as plsc

N = 10000
D = 128
E = 320000
EMB = 10
EMBP = 16

NC = 2
NS = 16
LANES = 16

B_E = 128
R_TOT = 2560
E_PAD = R_TOT * B_E
C0_RPT = 80
C1_RPT = 80
C1_BASE = NS * C0_RPT
CHUNK = 40
C0_CHUNKS = C0_RPT // CHUNK
NB_PAIRS = CHUNK // 2

ROW_BLK = 400
GRID = N // ROW_BLK

WB_CHUNK = 80
WB_NCHUNK = N // WB_CHUNK


def _prep_body(alpha_ref, x_ref, w0_ref, w1_ref, w2_ref, e2_ref,
               f0_ref, f1_ref, f2_ref, m_ref, acc_ref):
    i = pl.program_id(0)
    x = x_ref[...]
    dot = functools.partial(jnp.dot, preferred_element_type=jnp.float32,
                            precision=lax.Precision.HIGHEST)
    f0_ref[...] = dot(x, w0_ref[...])
    f1_ref[...] = dot(x, w1_ref[...])
    f2_ref[...] = dot(x, w2_ref[...])
    contrib = lax.dot_general(e2_ref[...], x, (((0,), (0,)), ((), ())),
                              preferred_element_type=jnp.float32,
                              precision=lax.Precision.HIGHEST)

    @pl.when(i == 0)
    def _():
        acc_ref[...] = contrib

    @pl.when(i != 0)
    def _():
        acc_ref[...] = acc_ref[...] + contrib

    @pl.when(i == GRID - 1)
    def _():
        wsum = w0_ref[...] + w1_ref[...] + w2_ref[...]
        m_ref[...] = alpha_ref[0] * dot(acc_ref[...], wsum)


def _dense_prep(x, w0, w1, w2, e2p, alpha):
    alpha1 = jnp.reshape(alpha, (1,))
    return pl.pallas_call(
        _prep_body,
        grid=(GRID,),
        in_specs=[
            pl.BlockSpec(memory_space=pltpu.SMEM),
            pl.BlockSpec((ROW_BLK, D), lambda i: (i, 0)),
            pl.BlockSpec((D, D), lambda i: (0, 0)),
            pl.BlockSpec((D, D), lambda i: (0, 0)),
            pl.BlockSpec((D, D), lambda i: (0, 0)),
            pl.BlockSpec((ROW_BLK, EMBP), lambda i: (i, 0)),
        ],
        out_specs=[
            pl.BlockSpec((ROW_BLK, D), lambda i: (i, 0)),
            pl.BlockSpec((ROW_BLK, D), lambda i: (i, 0)),
            pl.BlockSpec((ROW_BLK, D), lambda i: (i, 0)),
            pl.BlockSpec((EMBP, D), lambda i: (0, 0)),
        ],
        out_shape=[
            jax.ShapeDtypeStruct((N, D), jnp.float32),
            jax.ShapeDtypeStruct((N, D), jnp.float32),
            jax.ShapeDtypeStruct((N, D), jnp.float32),
            jax.ShapeDtypeStruct((EMBP, D), jnp.float32),
        ],
        scratch_shapes=[pltpu.VMEM((EMBP, D), jnp.float32)],
    )(alpha1, x, w0, w1, w2, e2p)


_MESH = plsc.VectorSubcoreMesh(core_axis_name="c", subcore_axis_name="s")


@functools.partial(
    pl.kernel,
    out_type=jax.ShapeDtypeStruct((NC, N, D), jnp.float32),
    mesh=_MESH,
    scratch_types=[
        pltpu.VMEM_SHARED((N, D), jnp.float32),
        pltpu.VMEM((CHUNK, B_E), jnp.int32),
        pltpu.VMEM((CHUNK, B_E), jnp.int32),
        pltpu.VMEM((CHUNK, B_E), jnp.float32),
        pltpu.VMEM((B_E, D), jnp.float32),
        pltpu.VMEM((B_E, D), jnp.float32),
        pltpu.SemaphoreType.DMA,
        pltpu.SemaphoreType.DMA,
        pltpu.SemaphoreType.DMA,
        pltpu.SemaphoreType.DMA,
    ],
)
def _sc_segment(f0, f1, f2, es0, ed0, es1, ed1, es2, ed2, ew0, ew1, ew2,
                out_hbm, acc_sh, src2d, dst2d, w2d, rows_a, rows_b,
                sem_ga, sem_gb, sem_sa, sem_sb):
    c = lax.axis_index("c")
    s = lax.axis_index("s")

    zvec = jnp.zeros((LANES,), jnp.float32)

    @pl.loop(0, B_E)
    def _(r):
        for cc in range(D // LANES):
            rows_a[r, pl.ds(cc * LANES, LANES)] = zvec

    for k in range((WB_NCHUNK + NS - 1) // NS):
        cid = s + NS * k

        @pl.when(cid < WB_NCHUNK)
        def _():
            pltpu.sync_copy(rows_a.at[pl.ds(0, WB_CHUNK)],
                            acc_sh.at[pl.ds(cid * WB_CHUNK, WB_CHUNK)])
    plsc.subcore_barrier()

    def scale(rows_v, j):
        @pl.loop(0, B_E // LANES)
        def _(g):
            wgrp = w2d[j, pl.ds(g * LANES, LANES)]
            for r in range(LANES):
                wvec = jnp.full((LANES,), wgrp[r], jnp.float32)
                row = g * LANES + r
                for cc in range(D // LANES):
                    sl = (row, pl.ds(cc * LANES, LANES))
                    rows_v[sl] = rows_v[sl] * wvec

    base = (1 - c) * (s * C0_RPT) + c * (C1_BASE + s * C1_RPT)
    for f, es, ed, ew in ((f0, es0, ed0, ew0), (f1, es1, ed1, ew1),
                          (f2, es2, ed2, ew2)):
        for chunk in range(C0_CHUNKS):
            row0 = base + chunk * CHUNK
            pltpu.sync_copy(es.at[pl.ds(row0, CHUNK)], src2d)
            pltpu.sync_copy(ed.at[pl.ds(row0, CHUNK)], dst2d)
            pltpu.sync_copy(ew.at[pl.ds(row0, CHUNK)], w2d)

            def gather(rows_v, sem, j):
                return pltpu.make_async_copy(f.at[src2d.at[j]], rows_v, sem)

            def scatter(rows_v, sem, j):
                return pltpu.make_async_copy(rows_v, acc_sh.at[dst2d.at[j]],
                                             sem)

            gather(rows_a, sem_ga, 0).start()
            gather(rows_b, sem_gb, 1).start()

            @pl.loop(0, NB_PAIRS)
            def _(p):
                j0 = 2 * p
                j1 = j0 + 1
                gather(rows_a, sem_ga, j0).wait()
                scale(rows_a, j0)
                scatter(rows_a, sem_sa, j0).start(add=True)
                gather(rows_b, sem_gb, j1).wait()
                scale(rows_b, j1)
                scatter(rows_b, sem_sb, j1).start(add=True)
                scatter(rows_a, sem_sa, j0).wait()
                scatter(rows_b, sem_sb, j1).wait()

                @pl.when(p < NB_PAIRS - 1)
                def _():
                    gather(rows_a, sem_ga, j0 + 2).start()
                    gather(rows_b, sem_gb, j1 + 2).start()

    plsc.subcore_barrier()
    for k in range((WB_NCHUNK + NS - 1) // NS):
        cid = s + NS * k

        @pl.when(cid < WB_NCHUNK)
        def _():
            sl = pl.ds(cid * WB_CHUNK, WB_CHUNK)
            pltpu.sync_copy(acc_sh.at[sl], out_hbm.at[c, sl])


def _final_body(acc_ref, e1_ref, m_ref, o_ref):
    learned = jnp.dot(e1_ref[...], m_ref[...],
                      preferred_element_type=jnp.float32,
                      precision=lax.Precision.HIGHEST)
    o_ref[...] = jnp.maximum(acc_ref[0] + acc_ref[1] + learned, 0.0)


def _finalize(acc, e1p, m):
    return pl.pallas_call(
        _final_body,
        grid=(GRID,),
        in_specs=[
            pl.BlockSpec((NC, ROW_BLK, D), lambda i: (0, i, 0)),
            pl.BlockSpec((ROW_BLK, EMBP), lambda i: (i, 0)),
            pl.BlockSpec((EMBP, D), lambda i: (0, 0)),
        ],
        out_specs=pl.BlockSpec((ROW_BLK, D), lambda i: (i, 0)),
        out_shape=jax.ShapeDtypeStruct((N, D), jnp.float32),
    )(acc, e1p, m)


def kernel(node_features, edge_index_0, edge_weight_0, edge_index_1,
           edge_weight_1, edge_index_2, edge_weight_2, W0, W1, W2,
           embed1, embed2, alpha):
    e1p = jnp.pad(embed1, ((0, 0), (0, EMBP - EMB)))
    e2p = jnp.pad(embed2, ((0, 0), (0, EMBP - EMB)))

    pad = E_PAD - E
    pad_idx = jnp.arange(pad, dtype=jnp.int32) % N

    def prep_edges(ei, ew):
        src = jnp.concatenate([ei[0], pad_idx]).reshape(E_PAD // B_E, B_E)
        dst = jnp.concatenate([ei[1], pad_idx]).reshape(E_PAD // B_E, B_E)
        w = jnp.pad(ew, (0, pad)).reshape(E_PAD // B_E, B_E)
        return src, dst, w

    s0, d0, w0 = prep_edges(edge_index_0, edge_weight_0)
    s1, d1, w1 = prep_edges(edge_index_1, edge_weight_1)
    s2, d2, w2 = prep_edges(edge_index_2, edge_weight_2)
    f0, f1, f2, m = _dense_prep(node_features, W0, W1, W2, e2p, alpha)
    acc = _sc_segment(f0, f1, f2, s0, d0, s1, d1, s2, d2, w0, w1, w2)
    return _finalize(acc, e1p, m)


# speedup vs baseline: 1.1464x; 1.1464x over previous
import functools

import jax
import jax.numpy as jnp
from jax import lax
from jax.experimental import pallas as pl
from jax.experimental.pallas import tpu as pltpu
from jax.experimental.pallas import tpu_sc as plsc

N = 10000
D = 128
E = 320000
EMB = 10
EMBP = 16

NC = 2
NS = 16
LANES = 16

B_E = 128
R_TOT = 2560
E_PAD = R_TOT * B_E
C0_RPT = 80
C1_RPT = 80
C1_BASE = NS * C0_RPT
CHUNK = 40
C0_CHUNKS = C0_RPT // CHUNK
NB_PAIRS = CHUNK // 2

ROW_BLK = 400
GRID = N // ROW_BLK

WB_CHUNK = 80
WB_NCHUNK = N // WB_CHUNK


def _prep_body(alpha_ref, x_ref, w0_ref, w1_ref, w2_ref, e2_ref,
               f0_ref, f1_ref, f2_ref, m_ref, acc_ref):
    i = pl.program_id(0)
    x = x_ref[...]
    dot = functools.partial(jnp.dot, preferred_element_type=jnp.float32,
                            precision=lax.Precision.HIGHEST)
    f0_ref[...] = dot(x, w0_ref[...])
    f1_ref[...] = dot(x, w1_ref[...])
    f2_ref[...] = dot(x, w2_ref[...])
    contrib = lax.dot_general(e2_ref[...], x, (((0,), (0,)), ((), ())),
                              preferred_element_type=jnp.float32,
                              precision=lax.Precision.HIGHEST)

    @pl.when(i == 0)
    def _():
        acc_ref[...] = contrib

    @pl.when(i != 0)
    def _():
        acc_ref[...] = acc_ref[...] + contrib

    @pl.when(i == GRID - 1)
    def _():
        wsum = w0_ref[...] + w1_ref[...] + w2_ref[...]
        m_ref[...] = alpha_ref[0] * dot(acc_ref[...], wsum)


def _dense_prep(x, w0, w1, w2, e2p, alpha):
    alpha1 = jnp.reshape(alpha, (1,))
    return pl.pallas_call(
        _prep_body,
        grid=(GRID,),
        in_specs=[
            pl.BlockSpec(memory_space=pltpu.SMEM),
            pl.BlockSpec((ROW_BLK, D), lambda i: (i, 0)),
            pl.BlockSpec((D, D), lambda i: (0, 0)),
            pl.BlockSpec((D, D), lambda i: (0, 0)),
            pl.BlockSpec((D, D), lambda i: (0, 0)),
            pl.BlockSpec((ROW_BLK, EMBP), lambda i: (i, 0)),
        ],
        out_specs=[
            pl.BlockSpec((ROW_BLK, D), lambda i: (i, 0)),
            pl.BlockSpec((ROW_BLK, D), lambda i: (i, 0)),
            pl.BlockSpec((ROW_BLK, D), lambda i: (i, 0)),
            pl.BlockSpec((EMBP, D), lambda i: (0, 0)),
        ],
        out_shape=[
            jax.ShapeDtypeStruct((N, D), jnp.float32),
            jax.ShapeDtypeStruct((N, D), jnp.float32),
            jax.ShapeDtypeStruct((N, D), jnp.float32),
            jax.ShapeDtypeStruct((EMBP, D), jnp.float32),
        ],
        scratch_shapes=[pltpu.VMEM((EMBP, D), jnp.float32)],
    )(alpha1, x, w0, w1, w2, e2p)


_MESH = plsc.VectorSubcoreMesh(core_axis_name="c", subcore_axis_name="s")


@functools.partial(
    pl.kernel,
    out_type=jax.ShapeDtypeStruct((NC, N, D), jnp.float32),
    mesh=_MESH,
    scratch_types=[
        pltpu.VMEM_SHARED((N, D), jnp.float32),
        pltpu.VMEM((CHUNK, B_E), jnp.int32),
        pltpu.VMEM((CHUNK, B_E), jnp.int32),
        pltpu.VMEM((CHUNK, B_E), jnp.float32),
        pltpu.VMEM((B_E, D), jnp.float32),
        pltpu.VMEM((B_E, D), jnp.float32),
        pltpu.SemaphoreType.DMA,
        pltpu.SemaphoreType.DMA,
        pltpu.SemaphoreType.DMA,
        pltpu.SemaphoreType.DMA,
    ],
)
def _sc_segment(f0, f1, f2, es0, ed0, es1, ed1, es2, ed2, ew0, ew1, ew2,
                out_hbm, acc_sh, src2d, dst2d, w2d, rows_a, rows_b,
                sem_ga, sem_gb, sem_sa, sem_sb):
    c = lax.axis_index("c")
    s = lax.axis_index("s")

    zvec = jnp.zeros((LANES,), jnp.float32)

    @pl.loop(0, B_E)
    def _(r):
        for cc in range(D // LANES):
            rows_a[r, pl.ds(cc * LANES, LANES)] = zvec

    for k in range((WB_NCHUNK + NS - 1) // NS):
        cid = s + NS * k

        @pl.when(cid < WB_NCHUNK)
        def _():
            pltpu.sync_copy(rows_a.at[pl.ds(0, WB_CHUNK)],
                            acc_sh.at[pl.ds(cid * WB_CHUNK, WB_CHUNK)])
    plsc.subcore_barrier()

    def scale(rows_v, j):
        @pl.loop(0, B_E // LANES)
        def _(g):
            wgrp = w2d[j, pl.ds(g * LANES, LANES)]
            for r in range(LANES):
                wvec = jnp.full((LANES,), wgrp[r], jnp.float32)
                row = g * LANES + r
                for cc in range(D // LANES):
                    sl = (row, pl.ds(cc * LANES, LANES))
                    rows_v[sl] = rows_v[sl] * wvec

    base = (1 - c) * (s * C0_RPT) + c * (C1_BASE + s * C1_RPT)
    for f, es, ed, ew in ((f0, es0, ed0, ew0), (f1, es1, ed1, ew1),
                          (f2, es2, ed2, ew2)):
        for chunk in range(C0_CHUNKS):
            row0 = base + chunk * CHUNK
            pltpu.sync_copy(es.at[pl.ds(row0, CHUNK)], src2d)
            pltpu.sync_copy(ed.at[pl.ds(row0, CHUNK)], dst2d)
            pltpu.sync_copy(ew.at[pl.ds(row0, CHUNK)], w2d)

            def gather(rows_v, sem, j):
                return pltpu.make_async_copy(f.at[src2d.at[j]], rows_v, sem)

            def scatter(rows_v, sem, j):
                return pltpu.make_async_copy(rows_v, acc_sh.at[dst2d.at[j]],
                                             sem)

            gather(rows_a, sem_ga, 0).start()

            @pl.loop(0, NB_PAIRS)
            def _(p):
                j0 = 2 * p
                j1 = j0 + 1
                gather(rows_b, sem_gb, j1).start()
                gather(rows_a, sem_ga, j0).wait()
                scale(rows_a, j0)
                scatter(rows_a, sem_sa, j0).start(add=True)
                gather(rows_b, sem_gb, j1).wait()
                scale(rows_b, j1)
                scatter(rows_b, sem_sb, j1).start(add=True)
                scatter(rows_a, sem_sa, j0).wait()

                @pl.when(p < NB_PAIRS - 1)
                def _():
                    gather(rows_a, sem_ga, j0 + 2).start()
                scatter(rows_b, sem_sb, j1).wait()

    plsc.subcore_barrier()
    for k in range((WB_NCHUNK + NS - 1) // NS):
        cid = s + NS * k

        @pl.when(cid < WB_NCHUNK)
        def _():
            sl = pl.ds(cid * WB_CHUNK, WB_CHUNK)
            pltpu.sync_copy(acc_sh.at[sl], out_hbm.at[c, sl])


def _final_body(acc_ref, e1_ref, m_ref, o_ref):
    learned = jnp.dot(e1_ref[...], m_ref[...],
                      preferred_element_type=jnp.float32,
                      precision=lax.Precision.HIGHEST)
    o_ref[...] = jnp.maximum(acc_ref[0] + acc_ref[1] + learned, 0.0)


def _finalize(acc, e1p, m):
    return pl.pallas_call(
        _final_body,
        grid=(GRID,),
        in_specs=[
            pl.BlockSpec((NC, ROW_BLK, D), lambda i: (0, i, 0)),
            pl.BlockSpec((ROW_BLK, EMBP), lambda i: (i, 0)),
            pl.BlockSpec((EMBP, D), lambda i: (0, 0)),
        ],
        out_specs=pl.BlockSpec((ROW_BLK, D), lambda i: (i, 0)),
        out_shape=jax.ShapeDtypeStruct((N, D), jnp.float32),
    )(acc, e1p, m)


def kernel(node_features, edge_index_0, edge_weight_0, edge_index_1,
           edge_weight_1, edge_index_2, edge_weight_2, W0, W1, W2,
           embed1, embed2, alpha):
    e1p = jnp.pad(embed1, ((0, 0), (0, EMBP - EMB)))
    e2p = jnp.pad(embed2, ((0, 0), (0, EMBP - EMB)))

    pad = E_PAD - E
    pad_idx = jnp.arange(pad, dtype=jnp.int32) % N

    def prep_edges(ei, ew):
        src = jnp.concatenate([ei[0], pad_idx]).reshape(E_PAD // B_E, B_E)
        dst = jnp.concatenate([ei[1], pad_idx]).reshape(E_PAD // B_E, B_E)
        w = jnp.pad(ew, (0, pad)).reshape(E_PAD // B_E, B_E)
        return src, dst, w

    s0, d0, w0 = prep_edges(edge_index_0, edge_weight_0)
    s1, d1, w1 = prep_edges(edge_index_1, edge_weight_1)
    s2, d2, w2 = prep_edges(edge_index_2, edge_weight_2)
    f0, f1, f2, m = _dense_prep(node_features, W0, W1, W2, e2p, alpha)
    acc = _sc_segment(f0, f1, f2, s0, d0, s1, d1, s2, d2, w0, w1, w2)
    return _finalize(acc, e1p, m)


# speedup vs baseline: 1.2815x; 1.1178x over previous
import functools

import jax
import jax.numpy as jnp
from jax import lax
from jax.experimental import pallas as pl
from jax.experimental.pallas import tpu as pltpu
from jax.experimental.pallas import tpu_sc as plsc

N = 10000
D = 128
E = 320000
EMB = 10
EMBP = 16

NC = 2
NS = 16
LANES = 16

B_E = 128
R_TOT = 2560
E_PAD = R_TOT * B_E
C0_RPT = 80
C1_RPT = 80
C1_BASE = NS * C0_RPT
CHUNK = 40
C0_CHUNKS = C0_RPT // CHUNK
NB_PAIRS = CHUNK // 2

ROW_BLK = 400
GRID = N // ROW_BLK

WB_CHUNK = 80
WB_NCHUNK = N // WB_CHUNK


def _prep_body(alpha_ref, x_ref, w0_ref, w1_ref, w2_ref, e2_ref,
               f0_ref, f1_ref, f2_ref, m_ref, acc_ref):
    i = pl.program_id(0)
    x = x_ref[...]
    dot = functools.partial(jnp.dot, preferred_element_type=jnp.float32,
                            precision=lax.Precision.HIGHEST)
    fdot = functools.partial(jnp.dot, preferred_element_type=jnp.float32)
    f0_ref[...] = fdot(x, w0_ref[...])
    f1_ref[...] = fdot(x, w1_ref[...])
    f2_ref[...] = fdot(x, w2_ref[...])
    contrib = lax.dot_general(e2_ref[...], x, (((0,), (0,)), ((), ())),
                              preferred_element_type=jnp.float32,
                              precision=lax.Precision.HIGHEST)

    @pl.when(i == 0)
    def _():
        acc_ref[...] = contrib

    @pl.when(i != 0)
    def _():
        acc_ref[...] = acc_ref[...] + contrib

    @pl.when(i == GRID - 1)
    def _():
        wsum = w0_ref[...] + w1_ref[...] + w2_ref[...]
        m_ref[...] = alpha_ref[0] * dot(acc_ref[...], wsum)


def _dense_prep(x, w0, w1, w2, e2p, alpha):
    alpha1 = jnp.reshape(alpha, (1,))
    return pl.pallas_call(
        _prep_body,
        grid=(GRID,),
        in_specs=[
            pl.BlockSpec(memory_space=pltpu.SMEM),
            pl.BlockSpec((ROW_BLK, D), lambda i: (i, 0)),
            pl.BlockSpec((D, D), lambda i: (0, 0)),
            pl.BlockSpec((D, D), lambda i: (0, 0)),
            pl.BlockSpec((D, D), lambda i: (0, 0)),
            pl.BlockSpec((ROW_BLK, EMBP), lambda i: (i, 0)),
        ],
        out_specs=[
            pl.BlockSpec((ROW_BLK, D), lambda i: (i, 0)),
            pl.BlockSpec((ROW_BLK, D), lambda i: (i, 0)),
            pl.BlockSpec((ROW_BLK, D), lambda i: (i, 0)),
            pl.BlockSpec((EMBP, D), lambda i: (0, 0)),
        ],
        out_shape=[
            jax.ShapeDtypeStruct((N, D), jnp.float32),
            jax.ShapeDtypeStruct((N, D), jnp.float32),
            jax.ShapeDtypeStruct((N, D), jnp.float32),
            jax.ShapeDtypeStruct((EMBP, D), jnp.float32),
        ],
        scratch_shapes=[pltpu.VMEM((EMBP, D), jnp.float32)],
    )(alpha1, x, w0, w1, w2, e2p)


_MESH = plsc.VectorSubcoreMesh(core_axis_name="c", subcore_axis_name="s")


@functools.partial(
    pl.kernel,
    out_type=jax.ShapeDtypeStruct((NC, N, D), jnp.float32),
    mesh=_MESH,
    scratch_types=[
        pltpu.VMEM_SHARED((N, D), jnp.float32),
        pltpu.VMEM((CHUNK, B_E), jnp.int32),
        pltpu.VMEM((CHUNK, B_E), jnp.int32),
        pltpu.VMEM((CHUNK, B_E), jnp.float32),
        pltpu.VMEM((B_E, D), jnp.float32),
        pltpu.VMEM((B_E, D), jnp.float32),
        pltpu.SemaphoreType.DMA,
        pltpu.SemaphoreType.DMA,
    ],
)
def _sc_segment(f0, f1, f2, es0, ed0, es1, ed1, es2, ed2, ew0, ew1, ew2,
                out_hbm, acc_sh, src2d, dst2d, w2d, rows_a, rows_b,
                sem_ga, sem_gb):
    c = lax.axis_index("c")
    s = lax.axis_index("s")

    zvec = jnp.zeros((LANES,), jnp.float32)

    @pl.loop(0, B_E)
    def _(r):
        for cc in range(D // LANES):
            rows_a[r, pl.ds(cc * LANES, LANES)] = zvec

    for k in range((WB_NCHUNK + NS - 1) // NS):
        cid = s + NS * k

        @pl.when(cid < WB_NCHUNK)
        def _():
            pltpu.sync_copy(rows_a.at[pl.ds(0, WB_CHUNK)],
                            acc_sh.at[pl.ds(cid * WB_CHUNK, WB_CHUNK)])
    plsc.subcore_barrier()

    def scale(rows_v, j):
        @pl.loop(0, B_E // LANES)
        def _(g):
            wgrp = w2d[j, pl.ds(g * LANES, LANES)]
            for r in range(LANES):
                wvec = jnp.full((LANES,), wgrp[r], jnp.float32)
                row = g * LANES + r
                for cc in range(D // LANES):
                    sl = (row, pl.ds(cc * LANES, LANES))
                    rows_v[sl] = rows_v[sl] * wvec

    base = (1 - c) * (s * C0_RPT) + c * (C1_BASE + s * C1_RPT)
    for f, es, ed, ew in ((f0, es0, ed0, ew0), (f1, es1, ed1, ew1),
                          (f2, es2, ed2, ew2)):
        for chunk in range(C0_CHUNKS):
            row0 = base + chunk * CHUNK
            pltpu.sync_copy(es.at[pl.ds(row0, CHUNK)], src2d)
            pltpu.sync_copy(ed.at[pl.ds(row0, CHUNK)], dst2d)
            pltpu.sync_copy(ew.at[pl.ds(row0, CHUNK)], w2d)

            def gather(rows_v, sem, j):
                return pltpu.make_async_copy(f.at[src2d.at[j]], rows_v, sem)

            gather(rows_a, sem_ga, 0).start()

            @pl.loop(0, NB_PAIRS)
            def _(p):
                j0 = 2 * p
                j1 = j0 + 1
                gather(rows_b, sem_gb, j1).start()
                gather(rows_a, sem_ga, j0).wait()
                scale(rows_a, j0)
                pltpu.sync_copy(rows_a, acc_sh.at[dst2d.at[j0]], add=True)

                @pl.when(p < NB_PAIRS - 1)
                def _():
                    gather(rows_a, sem_ga, j0 + 2).start()
                gather(rows_b, sem_gb, j1).wait()
                scale(rows_b, j1)
                pltpu.sync_copy(rows_b, acc_sh.at[dst2d.at[j1]], add=True)

    plsc.subcore_barrier()
    for k in range((WB_NCHUNK + NS - 1) // NS):
        cid = s + NS * k

        @pl.when(cid < WB_NCHUNK)
        def _():
            sl = pl.ds(cid * WB_CHUNK, WB_CHUNK)
            pltpu.sync_copy(acc_sh.at[sl], out_hbm.at[c, sl])


def _final_body(acc_ref, e1_ref, m_ref, o_ref):
    learned = jnp.dot(e1_ref[...], m_ref[...],
                      preferred_element_type=jnp.float32,
                      precision=lax.Precision.HIGHEST)
    o_ref[...] = jnp.maximum(acc_ref[0] + acc_ref[1] + learned, 0.0)


def _finalize(acc, e1p, m):
    return pl.pallas_call(
        _final_body,
        grid=(GRID,),
        in_specs=[
            pl.BlockSpec((NC, ROW_BLK, D), lambda i: (0, i, 0)),
            pl.BlockSpec((ROW_BLK, EMBP), lambda i: (i, 0)),
            pl.BlockSpec((EMBP, D), lambda i: (0, 0)),
        ],
        out_specs=pl.BlockSpec((ROW_BLK, D), lambda i: (i, 0)),
        out_shape=jax.ShapeDtypeStruct((N, D), jnp.float32),
    )(acc, e1p, m)


def kernel(node_features, edge_index_0, edge_weight_0, edge_index_1,
           edge_weight_1, edge_index_2, edge_weight_2, W0, W1, W2,
           embed1, embed2, alpha):
    e1p = jnp.pad(embed1, ((0, 0), (0, EMBP - EMB)))
    e2p = jnp.pad(embed2, ((0, 0), (0, EMBP - EMB)))

    pad = E_PAD - E
    pad_idx = jnp.arange(pad, dtype=jnp.int32) % N

    def prep_edges(ei, ew):
        src = jnp.concatenate([ei[0], pad_idx]).reshape(E_PAD // B_E, B_E)
        dst = jnp.concatenate([ei[1], pad_idx]).reshape(E_PAD // B_E, B_E)
        w = jnp.pad(ew, (0, pad)).reshape(E_PAD // B_E, B_E)
        return src, dst, w

    s0, d0, w0 = prep_edges(edge_index_0, edge_weight_0)
    s1, d1, w1 = prep_edges(edge_index_1, edge_weight_1)
    s2, d2, w2 = prep_edges(edge_index_2, edge_weight_2)
    f0, f1, f2, m = _dense_prep(node_features, W0, W1, W2, e2p, alpha)
    acc = _sc_segment(f0, f1, f2, s0, d0, s1, d1, s2, d2, w0, w1, w2)
    return _finalize(acc, e1p, m)
